# Initial kernel scaffold; baseline (speedup 1.0000x reference)
import jax
import jax.numpy as jnp
from jax.experimental import pallas as pl


def kernel(x, edge_index, W_gcn, b_gcn, ln_gamma, ln_beta, Ws1, Ws2, gate_W, gate_b, cls_W1, cls_b1, cls_W2, cls_b2):
    raise NotImplementedError("write your pallas kernel here")


# speedup vs baseline: 4.7949x; 4.7949x over previous
import dataclasses

import jax
import jax.numpy as jnp
from jax import lax
from jax.experimental import pallas as pl
from jax.experimental.pallas import tpu as pltpu
from jax.experimental.pallas import tpu_sc as plsc

_N = 10000
_D = 128
_A = 64
_R = 2
_E = 160000
_TOPK = 10

_NUM_SC = 2
_NUM_TILES = 16
_LANES = 16

_vec_mesh = plsc.VectorSubcoreMesh(core_axis_name="c", subcore_axis_name="s")

_sc_params = pltpu.CompilerParams()
if "needs_layout_passes" in pltpu.CompilerParams.__dataclass_fields__:
    _sc_params = dataclasses.replace(_sc_params, needs_layout_passes=False)

_DEG_PART = _E // 8
_DEG_CHUNK = 2000


def _deg_body(idx_hbm, out_hbm, deg_v, idxbuf_v):
    c = lax.axis_index("c")
    s = lax.axis_index("s")
    w = c * _NUM_TILES + s
    lst = w // 8
    part = w % 8

    @pl.loop(0, _N, step=_LANES)
    def _zero(i):
        deg_v[pl.ds(i, _LANES)] = jnp.zeros((_LANES,), jnp.float32)

    ones = jnp.ones((_LANES,), jnp.float32)

    @pl.loop(0, _DEG_PART, step=_DEG_CHUNK)
    def _chunk(cb):
        base = pl.multiple_of(lst * _E + part * _DEG_PART + cb, 8)
        pltpu.sync_copy(idx_hbm.at[pl.ds(base, _DEG_CHUNK)], idxbuf_v)

        @pl.loop(0, _DEG_CHUNK, step=_LANES)
        def _grp(gi):
            iv = idxbuf_v[pl.ds(gi, _LANES)]
            plsc.addupdate_scatter(deg_v, [iv], ones)

    pltpu.sync_copy(deg_v, out_hbm.at[w, 0])


def _degrees(idx_flat):
    return pl.kernel(
        _deg_body,
        out_type=jax.ShapeDtypeStruct((4 * 8, 1, _N), jnp.float32),
        mesh=_vec_mesh,
        compiler_params=_sc_params,
        scratch_types=[
            pltpu.VMEM((_N,), jnp.float32),
            pltpu.VMEM((_DEG_CHUNK,), jnp.int32),
        ],
    )(idx_flat)


def _bf16dot(a, b):
    return lax.dot_general(a.astype(jnp.bfloat16), b.astype(jnp.bfloat16),
                           (((1,), (0,)), ((), ())),
                           preferred_element_type=jnp.float32)


def _feat_body(x_ref, parts_ref, feat_ref, ndst_ref):
    parts = parts_ref[...]
    row = lax.broadcasted_iota(jnp.int32, (32, 4), 0) // 8
    col = lax.broadcasted_iota(jnp.int32, (32, 4), 1)
    sel = (row == col).astype(jnp.float32)
    degmat = lax.dot_general(parts, sel, (((0,), (0,)), ((), ())),
                             precision=lax.Precision.HIGHEST,
                             preferred_element_type=jnp.float32)
    norms = jnp.where(degmat > 0, lax.rsqrt(degmat), 0.0)
    xv = x_ref[...]
    feat_ref[0:_N, :] = xv * norms[:, 0:1]
    feat_ref[_N:2 * _N, :] = xv * norms[:, 2:3]
    ndst_ref[...] = jnp.concatenate([norms[:, 1:2], norms[:, 3:4]], axis=1)


def _feat(x, parts):
    return pl.pallas_call(
        _feat_body,
        out_shape=[
            jax.ShapeDtypeStruct((_R * _N, _D), jnp.float32),
            jax.ShapeDtypeStruct((_N, _R), jnp.float32),
        ],
    )(x, parts)


_CHUNK = 128
_NCHUNK = _E // _CHUNK
_ITERS = (_NCHUNK + _NUM_TILES - 1) // _NUM_TILES
_SLAB = 632
_NPAD = _SLAB * _NUM_TILES


def _agg_body(feat_hbm, idx_hbm, out_hbm, rows_v, srcbuf_v, dstbuf_v, acc_sh):
    c = lax.axis_index("c")
    s = lax.axis_index("s")

    @pl.loop(0, _CHUNK)
    def _zr(r):
        for k in range(_D // _LANES):
            rows_v[r, pl.ds(k * _LANES, _LANES)] = jnp.zeros((_LANES,),
                                                             jnp.float32)

    row0 = pl.multiple_of(s * _SLAB, 8)
    for j in range(_SLAB // _CHUNK):
        pltpu.sync_copy(rows_v, acc_sh.at[pl.ds(row0 + j * _CHUNK, _CHUNK)])
    rem = _SLAB % _CHUNK
    if rem:
        pltpu.sync_copy(
            rows_v.at[pl.ds(0, rem)],
            acc_sh.at[pl.ds(row0 + (_SLAB // _CHUNK) * _CHUNK, rem)])
    plsc.subcore_barrier()

    noff = c * _N
    w = c * _NUM_TILES + s

    @pl.loop(0, _ITERS)
    def _it(i):
        cid = i * _NUM_TILES + s

        @pl.when(cid < _NCHUNK)
        def _():
            sbase = pl.multiple_of(2 * c * _E + cid * _CHUNK, 8)
            dbase = pl.multiple_of((2 * c + 1) * _E + cid * _CHUNK, 8)
            pltpu.sync_copy(idx_hbm.at[pl.ds(sbase, _CHUNK)], srcbuf_v)
            pltpu.sync_copy(idx_hbm.at[pl.ds(dbase, _CHUNK)], dstbuf_v)
            for k in range(_CHUNK // _LANES):
                sl = pl.ds(k * _LANES, _LANES)
                srcbuf_v[sl] = srcbuf_v[sl] + noff
            pltpu.sync_copy(feat_hbm.at[srcbuf_v], rows_v)
            pltpu.sync_copy(rows_v, acc_sh.at[dstbuf_v], add=True)

    plsc.subcore_barrier()
    pltpu.sync_copy(acc_sh.at[pl.ds(row0, _SLAB)], out_hbm.at[w])


def _aggregate(feat, idx_flat):
    return pl.kernel(
        _agg_body,
        out_type=jax.ShapeDtypeStruct((_R * _NUM_TILES, _SLAB, _D),
                                      jnp.float32),
        mesh=_vec_mesh,
        scratch_types=[
            pltpu.VMEM((_CHUNK, _D), jnp.float32),
            pltpu.VMEM((_CHUNK,), jnp.int32),
            pltpu.VMEM((_CHUNK,), jnp.int32),
            pltpu.VMEM_SHARED((_NPAD, _D), jnp.float32),
        ],
    )(feat, idx_flat)


_BLK = 1000


def _lane_sum128(v):
    g = v[:, 0:8]
    for k in range(1, 16):
        g = g + v[:, 8 * k:8 * (k + 1)]
    s04 = g[:, 0:1] + g[:, 4:5]
    s26 = g[:, 2:3] + g[:, 6:7]
    s15 = g[:, 1:2] + g[:, 5:6]
    s37 = g[:, 3:4] + g[:, 7:8]
    return (s04 + s26) + (s15 + s37)


def _gconv_body(agg_ref, ndst_ref, w_ref, b_ref, rst_ref):
    nd = ndst_ref[...]
    w = w_ref[...]
    b = b_ref[...]
    for r in range(_R):
        rst_ref[r] = _bf16dot(agg_ref[r] * nd[:, r:r + 1], w) + b


def _gconv(agg3, ndst, W_gcn, b_gcn):
    nblk = _N // _BLK
    return pl.pallas_call(
        _gconv_body,
        grid=(nblk,),
        in_specs=[
            pl.BlockSpec((_R, _BLK, _D), lambda i: (0, i, 0)),
            pl.BlockSpec((_BLK, _R), lambda i: (i, 0)),
            pl.BlockSpec((_D, _D), lambda i: (0, 0)),
            pl.BlockSpec((_D,), lambda i: (0,)),
        ],
        out_specs=[pl.BlockSpec((_R, _BLK, _D), lambda i: (0, i, 0))],
        out_shape=[jax.ShapeDtypeStruct((_R, _N, _D), jnp.float32)],
    )(agg3, ndst, W_gcn, b_gcn)[0]


def _node_body(h_ref, g_ref, be_ref, ws1_ref, ws2_ref, gw_ref, gb_ref,
               hcomb_ref, gate_ref):
    gamma = g_ref[...]
    beta = be_ref[...]
    hs = []
    logits = []
    for r in range(_R):
        v = h_ref[r]
        mu = _lane_sum128(v) * (1.0 / _D)
        dvi = v - mu
        var = _lane_sum128(dvi * dvi) * (1.0 / _D)
        ve = var + 1e-5
        y = lax.rsqrt(ve)
        y = y * (1.5 - 0.5 * ve * y * y)
        h = (dvi * y) * gamma + beta
        hs.append(h)
        sv = _bf16dot(h, ws1_ref[r])
        sv = 1.0 / (1.0 + jnp.exp(-sv))
        logits.append(_bf16dot(sv, ws2_ref[r]))
    m = jnp.maximum(logits[0], logits[1])
    e0 = jnp.exp(logits[0] - m)
    e1 = jnp.exp(logits[1] - m)
    rden = 1.0 / (e0 + e1)
    hc = (e0 * rden) * hs[0] + (e1 * rden) * hs[1]
    hcomb_ref[...] = hc
    gate_ref[...] = _bf16dot(hc, gw_ref[...]) + gb_ref[...]


def _node_epilogue(h_elu, ln_gamma, ln_beta, Ws1, Ws2, gate_W, gate_b):
    nblk = _N // _BLK
    return pl.pallas_call(
        _node_body,
        grid=(nblk,),
        in_specs=[
            pl.BlockSpec((_R, _BLK, _D), lambda i: (0, i, 0)),
            pl.BlockSpec((_D,), lambda i: (0,)),
            pl.BlockSpec((_D,), lambda i: (0,)),
            pl.BlockSpec((_R, _D, _A), lambda i: (0, 0, 0)),
            pl.BlockSpec((_R, _A, 1), lambda i: (0, 0, 0)),
            pl.BlockSpec((_D, 1), lambda i: (0, 0)),
            pl.BlockSpec((1,), lambda i: (0,)),
        ],
        out_specs=[
            pl.BlockSpec((_BLK, _D), lambda i: (i, 0)),
            pl.BlockSpec((_BLK, 1), lambda i: (i, 0)),
        ],
        out_shape=[
            jax.ShapeDtypeStruct((_N, _D), jnp.float32),
            jax.ShapeDtypeStruct((_N, 1), jnp.float32),
        ],
    )(h_elu, ln_gamma, ln_beta, Ws1, Ws2, gate_W, gate_b)


_PBLK = 2000
_NPB = _N // _PBLK


def _pool_block_body(hc_ref, g_ref, gt_ref, bmax_ref, esum_ref, pgx_ref):
    g = g_ref[...]
    gt = jnp.transpose(g)
    gt_ref[...] = gt.reshape(1, 1, _PBLK)
    bm = jnp.max(gt)
    e = jnp.exp(g - bm)
    es = jnp.sum(e)
    pgx = jnp.sum(e * hc_ref[...], axis=0)
    bmax_ref[...] = jnp.reshape(bm, (1, 1, 1))
    esum_ref[...] = jnp.reshape(es, (1, 1, 1))
    pgx_ref[...] = pgx.reshape(1, 1, _D)


def _pool_blocks(h_comb, g):
    return pl.pallas_call(
        _pool_block_body,
        grid=(_NPB,),
        in_specs=[
            pl.BlockSpec((_PBLK, _D), lambda i: (i, 0)),
            pl.BlockSpec((_PBLK, 1), lambda i: (i, 0)),
        ],
        out_specs=[
            pl.BlockSpec((1, 1, _PBLK), lambda i: (i, 0, 0)),
            pl.BlockSpec((1, 1, 1), lambda i: (i, 0, 0)),
            pl.BlockSpec((1, 1, 1), lambda i: (i, 0, 0)),
            pl.BlockSpec((1, 1, _D), lambda i: (i, 0, 0)),
        ],
        out_shape=[
            jax.ShapeDtypeStruct((_NPB, 1, _PBLK), jnp.float32),
            jax.ShapeDtypeStruct((_NPB, 1, 1), jnp.float32),
            jax.ShapeDtypeStruct((_NPB, 1, 1), jnp.float32),
            jax.ShapeDtypeStruct((_NPB, 1, _D), jnp.float32),
        ],
    )(h_comb, g)


def _pool_merge_body(gt_ref, bmax_ref, esum_ref, pgx_ref, w1_ref, b1_ref,
                     w2_ref, b2_ref, logit_ref, topw_ref, topi_ref):
    g2 = gt_ref[...].reshape(_NPB, _PBLK)
    bm = bmax_ref[...].reshape(_NPB, 1)
    es = esum_ref[...].reshape(_NPB, 1)
    m = jnp.max(bm)
    scale = jnp.exp(bm - m)
    rz = 1.0 / jnp.sum(es * scale)
    pg = pgx_ref[...].reshape(_NPB, _D)
    gx = jnp.sum(pg * scale, axis=0) * rz
    rf = _bf16dot(gx[None, :], w1_ref[...]) + b1_ref[...]
    rf = jnp.maximum(rf, 0.0)
    logit_ref[...] = _bf16dot(rf, w2_ref[...]) + b2_ref[...]

    row = lax.broadcasted_iota(jnp.int32, (_NPB, _PBLK), 0)
    col = lax.broadcasted_iota(jnp.int32, (_NPB, _PBLK), 1)
    ids = row * _PBLK + col
    kpos = lax.broadcasted_iota(jnp.int32, (1, _TOPK), 1)
    tw = jnp.zeros((1, _TOPK), jnp.float32)
    ti = jnp.zeros((1, _TOPK), jnp.int32)
    wts = jnp.exp(g2 - m) * rz
    for k in range(_TOPK):
        mk = jnp.max(wts)
        ik = jnp.min(jnp.where(wts == mk, ids, _N))
        tw = jnp.where(kpos == k, mk, tw)
        ti = jnp.where(kpos == k, ik, ti)
        wts = jnp.where(ids == ik, -jnp.inf, wts)
    topw_ref[...] = tw
    topi_ref[...] = ti


def _pool_head(h_comb, g, cls_W1, cls_b1, cls_W2, cls_b2):
    gt, bmax, esum, pgx = _pool_blocks(h_comb, g)
    return pl.pallas_call(
        _pool_merge_body,
        out_shape=[
            jax.ShapeDtypeStruct((1, 1), jnp.float32),
            jax.ShapeDtypeStruct((1, _TOPK), jnp.float32),
            jax.ShapeDtypeStruct((1, _TOPK), jnp.int32),
        ],
    )(gt, bmax, esum, pgx, cls_W1, cls_b1, cls_W2, cls_b2)


def kernel(x, edge_index, W_gcn, b_gcn, ln_gamma, ln_beta, Ws1, Ws2, gate_W,
           gate_b, cls_W1, cls_b1, cls_W2, cls_b2):
    idx_flat = edge_index.astype(jnp.int32).reshape(4 * _E)
    parts = _degrees(idx_flat)
    feat, ndst = _feat(x, parts.reshape(4 * 8, _N))
    slabs = _aggregate(feat, idx_flat)
    agg3 = slabs.reshape(_R, _NUM_TILES * _SLAB, _D)[:, :_N, :]
    rst = _gconv(agg3, ndst, W_gcn, b_gcn)
    h_elu = jnp.where(rst > 0, rst,
                      jnp.expm1(jnp.where(rst > 0, 0.0, rst)))
    h_comb, g = _node_epilogue(h_elu, ln_gamma, ln_beta, Ws1, Ws2,
                               gate_W, gate_b)
    logit, topw, topi = _pool_head(h_comb, g, cls_W1, cls_b1, cls_W2, cls_b2)
    return (logit.reshape(1), topw.reshape(_TOPK), topi.reshape(_TOPK))
